# Initial kernel scaffold; baseline (speedup 1.0000x reference)
import jax
import jax.numpy as jnp
from jax.experimental import pallas as pl


def kernel(x, edge_index, W1, b1, W2, b2):
    raise NotImplementedError("write your pallas kernel here")


# speedup vs baseline: 14.7335x; 14.7335x over previous
import functools

import jax
import jax.numpy as jnp
from jax import lax
from jax.experimental import pallas as pl
from jax.experimental.pallas import tpu as pltpu
from jax.experimental.pallas import tpu_sc as plsc

N = 10000
D = 128
DC = 64
E = 320000
NC = 2
NS = 16
NW = NC * NS
CH = 80
DNCH = E // NW // CH
ANCH = E // NS // CH
RPS = 624
RPL = N - 15 * RPS
DW = 16
RB = 1000

_MESH = plsc.VectorSubcoreMesh(core_axis_name="c", subcore_axis_name="s")
_SC_PARAMS = pltpu.CompilerParams(use_tc_tiling_on_sc=False)


@functools.partial(
    pl.kernel,
    out_type=jax.ShapeDtypeStruct((NC, N, DW), jnp.float32),
    mesh=_MESH,
    scratch_types=[
        pltpu.VMEM((DNCH, CH), jnp.int32),
        pltpu.VMEM((CH, DW), jnp.float32),
        pltpu.VMEM((RPL, DW), jnp.float32),
        pltpu.VMEM_SHARED((N, DW), jnp.float32),
    ],
    compiler_params=_SC_PARAMS,
)
def _deg_kernel(dst_hbm, out_hbm, idx_v, ones_v, buf_v, acc_sh):
    c = lax.axis_index("c")
    s = lax.axis_index("s")
    wid = c * NS + s
    r0 = s * RPS

    one = jnp.full((16,), 1.0, jnp.float32)
    zero = jnp.zeros((16,), jnp.float32)

    def fill_ones(i, _):
        ones_v[i, :] = one
        return 0

    lax.fori_loop(0, CH, fill_ones, 0)

    def fill_zero(i, _):
        buf_v[i, :] = zero
        return 0

    lax.fori_loop(0, RPL, fill_zero, 0)

    @pl.when(s < NS - 1)
    def _():
        pltpu.sync_copy(buf_v.at[pl.ds(0, RPS)], acc_sh.at[pl.ds(r0, RPS)])

    @pl.when(s == NS - 1)
    def _():
        pltpu.sync_copy(buf_v, acc_sh.at[pl.ds(r0, RPL)])

    pltpu.sync_copy(dst_hbm.at[wid], idx_v)
    plsc.subcore_barrier()

    def body(j, _):
        pltpu.sync_copy(ones_v, acc_sh.at[idx_v.at[j]], add=True)
        return 0

    lax.fori_loop(0, DNCH, body, 0)

    plsc.subcore_barrier()

    @pl.when(s < NS - 1)
    def _():
        pltpu.sync_copy(acc_sh.at[pl.ds(r0, RPS)], buf_v.at[pl.ds(0, RPS)])
        pltpu.sync_copy(buf_v.at[pl.ds(0, RPS)], out_hbm.at[c, pl.ds(r0, RPS)])

    @pl.when(s == NS - 1)
    def _():
        pltpu.sync_copy(acc_sh.at[pl.ds(r0, RPL)], buf_v)
        pltpu.sync_copy(buf_v, out_hbm.at[c, pl.ds(r0, RPL)])


@functools.partial(
    pl.kernel,
    out_type=jax.ShapeDtypeStruct((NC, N, DC), jnp.float32),
    mesh=_MESH,
    scratch_types=[
        pltpu.VMEM((ANCH, CH), jnp.int32),
        pltpu.VMEM((ANCH, CH), jnp.int32),
        pltpu.VMEM((CH, DC), jnp.float32),
        pltpu.VMEM((RPL, DC), jnp.float32),
        pltpu.VMEM_SHARED((N, DC), jnp.float32),
        pltpu.SemaphoreType.DMA,
    ],
    compiler_params=_SC_PARAMS,
)
def _agg_kernel(hlo_hbm, hhi_hbm, src_hbm, dst_hbm, out_hbm,
                sidx_v, didx_v, rows_v, buf_v, acc_sh, gsem):
    c = lax.axis_index("c")
    s = lax.axis_index("s")
    r0 = s * RPS

    pltpu.sync_copy(src_hbm.at[s], sidx_v)
    pltpu.sync_copy(dst_hbm.at[s], didx_v)

    def init_from(h_ref):
        @pl.when(s < NS - 1)
        def _():
            pltpu.sync_copy(h_ref.at[pl.ds(r0, RPS)], buf_v.at[pl.ds(0, RPS)])
            pltpu.sync_copy(buf_v.at[pl.ds(0, RPS)], acc_sh.at[pl.ds(r0, RPS)])

        @pl.when(s == NS - 1)
        def _():
            pltpu.sync_copy(h_ref.at[pl.ds(r0, RPL)], buf_v)
            pltpu.sync_copy(buf_v, acc_sh.at[pl.ds(r0, RPL)])

    def edge_loop(h_ref):
        def body(j, _):
            pltpu.async_copy(h_ref.at[sidx_v.at[j]], rows_v, gsem).wait()
            pltpu.sync_copy(rows_v, acc_sh.at[didx_v.at[j]], add=True)
            return 0

        lax.fori_loop(0, ANCH, body, 0)

    @pl.when(c == 0)
    def _():
        init_from(hlo_hbm)

    @pl.when(c == 1)
    def _():
        init_from(hhi_hbm)

    plsc.subcore_barrier()

    @pl.when(c == 0)
    def _():
        edge_loop(hlo_hbm)

    @pl.when(c == 1)
    def _():
        edge_loop(hhi_hbm)

    plsc.subcore_barrier()

    @pl.when(s < NS - 1)
    def _():
        pltpu.sync_copy(acc_sh.at[pl.ds(r0, RPS)], buf_v.at[pl.ds(0, RPS)])
        pltpu.sync_copy(buf_v.at[pl.ds(0, RPS)], out_hbm.at[c, pl.ds(r0, RPS)])

    @pl.when(s == NS - 1)
    def _():
        pltpu.sync_copy(acc_sh.at[pl.ds(r0, RPL)], buf_v)
        pltpu.sync_copy(buf_v, out_hbm.at[c, pl.ds(r0, RPL)])


def _dinv_block(da_ref, db_ref):
    return lax.rsqrt(1.0 + da_ref[:, :1] + db_ref[:, :1])


def _tc1_body(x_ref, w_ref, da_ref, db_ref, olo_ref, ohi_ref):
    dinv = _dinv_block(da_ref, db_ref)
    res = jnp.dot(x_ref[...], w_ref[...],
                  preferred_element_type=jnp.float32) * dinv
    olo_ref[...] = res[:, :DC]
    ohi_ref[...] = res[:, DC:]


def _tc2_body(aa_ref, ab_ref, da_ref, db_ref, b_ref, w_ref, olo_ref, ohi_ref):
    dinv = _dinv_block(da_ref, db_ref)
    agg = jnp.concatenate([aa_ref[0], ab_ref[0]], axis=1)
    h = jnp.maximum(agg * dinv + b_ref[...], 0.0)
    res = jnp.dot(h, w_ref[...],
                  preferred_element_type=jnp.float32) * dinv
    olo_ref[...] = res[:, :DC]
    ohi_ref[...] = res[:, DC:]


def _tc3_body(aa_ref, ab_ref, da_ref, db_ref, b_ref, o_ref):
    dinv = _dinv_block(da_ref, db_ref)
    agg = jnp.concatenate([aa_ref[0], ab_ref[0]], axis=1)
    o_ref[...] = agg * dinv + b_ref[...]


def _tc1(x, W1, dega, degb):
    return pl.pallas_call(
        _tc1_body,
        grid=(N // RB,),
        in_specs=[
            pl.BlockSpec((RB, D), lambda i: (i, 0)),
            pl.BlockSpec((D, D), lambda i: (0, 0)),
            pl.BlockSpec((RB, DW), lambda i: (i, 0)),
            pl.BlockSpec((RB, DW), lambda i: (i, 0)),
        ],
        out_specs=[
            pl.BlockSpec((RB, DC), lambda i: (i, 0)),
            pl.BlockSpec((RB, DC), lambda i: (i, 0)),
        ],
        out_shape=[
            jax.ShapeDtypeStruct((N, DC), jnp.float32),
            jax.ShapeDtypeStruct((N, DC), jnp.float32),
        ],
    )(x, W1, dega, degb)


def _tc2(agg, dega, degb, b1, W2):
    return pl.pallas_call(
        _tc2_body,
        grid=(N // RB,),
        in_specs=[
            pl.BlockSpec((1, RB, DC), lambda i: (0, i, 0)),
            pl.BlockSpec((1, RB, DC), lambda i: (1, i, 0)),
            pl.BlockSpec((RB, DW), lambda i: (i, 0)),
            pl.BlockSpec((RB, DW), lambda i: (i, 0)),
            pl.BlockSpec((D,), lambda i: (0,)),
            pl.BlockSpec((D, D), lambda i: (0, 0)),
        ],
        out_specs=[
            pl.BlockSpec((RB, DC), lambda i: (i, 0)),
            pl.BlockSpec((RB, DC), lambda i: (i, 0)),
        ],
        out_shape=[
            jax.ShapeDtypeStruct((N, DC), jnp.float32),
            jax.ShapeDtypeStruct((N, DC), jnp.float32),
        ],
    )(agg, agg, dega, degb, b1, W2)


def _tc3(agg, dega, degb, b2):
    return pl.pallas_call(
        _tc3_body,
        grid=(N // RB,),
        in_specs=[
            pl.BlockSpec((1, RB, DC), lambda i: (0, i, 0)),
            pl.BlockSpec((1, RB, DC), lambda i: (1, i, 0)),
            pl.BlockSpec((RB, DW), lambda i: (i, 0)),
            pl.BlockSpec((RB, DW), lambda i: (i, 0)),
            pl.BlockSpec((D,), lambda i: (0,)),
        ],
        out_specs=pl.BlockSpec((RB, D), lambda i: (i, 0)),
        out_shape=jax.ShapeDtypeStruct((N, D), jnp.float32),
    )(agg, agg, dega, degb, b2)


def kernel(x, edge_index, W1, b1, W2, b2):
    ei = edge_index.astype(jnp.int32)
    dst_d = ei[1].reshape(NW, DNCH, CH)
    src_a = ei[0].reshape(NS, ANCH, CH)
    dst_a = ei[1].reshape(NS, ANCH, CH)

    dego = _deg_kernel(dst_d)
    dega, degb = dego[0], dego[1]

    h1lo, h1hi = _tc1(x, W1, dega, degb)
    a1 = _agg_kernel(h1lo, h1hi, src_a, dst_a)
    h2lo, h2hi = _tc2(a1, dega, degb, b1, W2)
    a2 = _agg_kernel(h2lo, h2hi, src_a, dst_a)
    return _tc3(a2, dega, degb, b2)


# speedup vs baseline: 20.1584x; 1.3682x over previous
import functools

import jax
import jax.numpy as jnp
from jax import lax
from jax.experimental import pallas as pl
from jax.experimental.pallas import tpu as pltpu
from jax.experimental.pallas import tpu_sc as plsc

N = 10000
D = 128
DC = 64
E = 320000
NC = 2
NS = 16
NW = NC * NS
CH = 80
DNCH = E // NW // CH
ANCH = E // NS // CH
RPS = 624
RPL = N - 15 * RPS
DW = 16
RB = 1000

_MESH = plsc.VectorSubcoreMesh(core_axis_name="c", subcore_axis_name="s")
_SC_PARAMS = pltpu.CompilerParams(use_tc_tiling_on_sc=False)


@functools.partial(
    pl.kernel,
    out_type=jax.ShapeDtypeStruct((NC, N, DW), jnp.float32),
    mesh=_MESH,
    scratch_types=[
        pltpu.VMEM((DNCH, CH), jnp.int32),
        pltpu.VMEM((CH, DW), jnp.float32),
        pltpu.VMEM((RPL, DW), jnp.float32),
        pltpu.VMEM_SHARED((N, DW), jnp.float32),
    ],
    compiler_params=_SC_PARAMS,
)
def _deg_kernel(dst_hbm, out_hbm, idx_v, ones_v, buf_v, acc_sh):
    c = lax.axis_index("c")
    s = lax.axis_index("s")
    wid = c * NS + s
    r0 = s * RPS

    one = jnp.full((16,), 1.0, jnp.float32)
    zero = jnp.zeros((16,), jnp.float32)

    def fill_ones(i, _):
        ones_v[i, :] = one
        return 0

    lax.fori_loop(0, CH, fill_ones, 0)

    def fill_zero(i, _):
        buf_v[i, :] = zero
        return 0

    lax.fori_loop(0, RPL, fill_zero, 0)

    @pl.when(s < NS - 1)
    def _():
        pltpu.sync_copy(buf_v.at[pl.ds(0, RPS)], acc_sh.at[pl.ds(r0, RPS)])

    @pl.when(s == NS - 1)
    def _():
        pltpu.sync_copy(buf_v, acc_sh.at[pl.ds(r0, RPL)])

    pltpu.sync_copy(dst_hbm.at[wid], idx_v)
    plsc.subcore_barrier()

    def body(j, _):
        pltpu.sync_copy(ones_v, acc_sh.at[idx_v.at[j]], add=True)
        return 0

    lax.fori_loop(0, DNCH, body, 0)

    plsc.subcore_barrier()

    @pl.when(s < NS - 1)
    def _():
        pltpu.sync_copy(acc_sh.at[pl.ds(r0, RPS)], buf_v.at[pl.ds(0, RPS)])
        pltpu.sync_copy(buf_v.at[pl.ds(0, RPS)], out_hbm.at[c, pl.ds(r0, RPS)])

    @pl.when(s == NS - 1)
    def _():
        pltpu.sync_copy(acc_sh.at[pl.ds(r0, RPL)], buf_v)
        pltpu.sync_copy(buf_v, out_hbm.at[c, pl.ds(r0, RPL)])


@functools.partial(
    pl.kernel,
    out_type=jax.ShapeDtypeStruct((NC, N, DC), jnp.float32),
    mesh=_MESH,
    scratch_types=[
        pltpu.VMEM((ANCH, CH), jnp.int32),
        pltpu.VMEM((ANCH, CH), jnp.int32),
        pltpu.VMEM((CH, DC), jnp.float32),
        pltpu.VMEM((CH, DC), jnp.float32),
        pltpu.VMEM_SHARED((N, DC), jnp.float32),
        pltpu.SemaphoreType.DMA,
        pltpu.SemaphoreType.DMA,
        pltpu.SemaphoreType.DMA,
        pltpu.SemaphoreType.DMA,
    ],
    compiler_params=_SC_PARAMS,
)
def _agg_kernel(hlo_hbm, hhi_hbm, src_hbm, dst_hbm, out_hbm,
                sidx_v, didx_v, rows0_v, rows1_v, acc_sh,
                g0, g1, s0, s1):
    c = lax.axis_index("c")
    s = lax.axis_index("s")
    r0 = s * RPS
    TL = RPS - 7 * CH

    pltpu.sync_copy(src_hbm.at[s], sidx_v)
    pltpu.sync_copy(dst_hbm.at[s], didx_v)

    def init_from(h_ref):
        def cp(t, _):
            rr = r0 + t * CH
            pltpu.sync_copy(h_ref.at[pl.ds(rr, CH)], rows0_v)
            pltpu.sync_copy(rows0_v, acc_sh.at[pl.ds(rr, CH)])
            return 0

        lax.fori_loop(0, 7, cp, 0)
        rr = r0 + 7 * CH

        @pl.when(s < NS - 1)
        def _():
            pltpu.sync_copy(h_ref.at[pl.ds(rr, TL)], rows0_v.at[pl.ds(0, TL)])
            pltpu.sync_copy(rows0_v.at[pl.ds(0, TL)], acc_sh.at[pl.ds(rr, TL)])

        @pl.when(s == NS - 1)
        def _():
            pltpu.sync_copy(h_ref.at[pl.ds(rr, CH)], rows0_v)
            pltpu.sync_copy(rows0_v, acc_sh.at[pl.ds(rr, CH)])

    def edge_loop(h_ref):
        npair = ANCH // 2
        pltpu.async_copy(h_ref.at[sidx_v.at[0]], rows0_v, g0)
        pltpu.async_copy(h_ref.at[sidx_v.at[1]], rows1_v, g1)

        def body(jj, _):
            j0 = 2 * jj
            j1 = j0 + 1
            pltpu.make_async_copy(h_ref.at[sidx_v.at[j0]], rows0_v, g0).wait()
            sc0 = pltpu.async_copy(rows0_v, acc_sh.at[didx_v.at[j0]], s0,
                                   add=True)
            pltpu.make_async_copy(h_ref.at[sidx_v.at[j1]], rows1_v, g1).wait()
            sc1 = pltpu.async_copy(rows1_v, acc_sh.at[didx_v.at[j1]], s1,
                                   add=True)
            sc0.wait()

            @pl.when(jj < npair - 1)
            def _():
                pltpu.async_copy(h_ref.at[sidx_v.at[j0 + 2]], rows0_v, g0)

            sc1.wait()

            @pl.when(jj < npair - 1)
            def _():
                pltpu.async_copy(h_ref.at[sidx_v.at[j1 + 2]], rows1_v, g1)

            return 0

        lax.fori_loop(0, npair, body, 0)

    @pl.when(c == 0)
    def _():
        init_from(hlo_hbm)

    @pl.when(c == 1)
    def _():
        init_from(hhi_hbm)

    plsc.subcore_barrier()

    @pl.when(c == 0)
    def _():
        edge_loop(hlo_hbm)

    @pl.when(c == 1)
    def _():
        edge_loop(hhi_hbm)

    plsc.subcore_barrier()

    def wo(t, _):
        rr = r0 + t * CH
        pltpu.sync_copy(acc_sh.at[pl.ds(rr, CH)], rows0_v)
        pltpu.sync_copy(rows0_v, out_hbm.at[c, pl.ds(rr, CH)])
        return 0

    lax.fori_loop(0, 7, wo, 0)
    rw = r0 + 7 * CH

    @pl.when(s < NS - 1)
    def _():
        pltpu.sync_copy(acc_sh.at[pl.ds(rw, TL)], rows0_v.at[pl.ds(0, TL)])
        pltpu.sync_copy(rows0_v.at[pl.ds(0, TL)], out_hbm.at[c, pl.ds(rw, TL)])

    @pl.when(s == NS - 1)
    def _():
        pltpu.sync_copy(acc_sh.at[pl.ds(rw, CH)], rows0_v)
        pltpu.sync_copy(rows0_v, out_hbm.at[c, pl.ds(rw, CH)])


def _dinv_block(da_ref, db_ref):
    return lax.rsqrt(1.0 + da_ref[:, :1] + db_ref[:, :1])


def _tc1_body(x_ref, w_ref, da_ref, db_ref, olo_ref, ohi_ref):
    dinv = _dinv_block(da_ref, db_ref)
    res = jnp.dot(x_ref[...], w_ref[...],
                  preferred_element_type=jnp.float32) * dinv
    olo_ref[...] = res[:, :DC]
    ohi_ref[...] = res[:, DC:]


def _tc2_body(aa_ref, ab_ref, da_ref, db_ref, b_ref, w_ref, olo_ref, ohi_ref):
    dinv = _dinv_block(da_ref, db_ref)
    agg = jnp.concatenate([aa_ref[0], ab_ref[0]], axis=1)
    h = jnp.maximum(agg * dinv + b_ref[...], 0.0)
    res = jnp.dot(h, w_ref[...],
                  preferred_element_type=jnp.float32) * dinv
    olo_ref[...] = res[:, :DC]
    ohi_ref[...] = res[:, DC:]


def _tc3_body(aa_ref, ab_ref, da_ref, db_ref, b_ref, o_ref):
    dinv = _dinv_block(da_ref, db_ref)
    agg = jnp.concatenate([aa_ref[0], ab_ref[0]], axis=1)
    o_ref[...] = agg * dinv + b_ref[...]


def _tc1(x, W1, dega, degb):
    return pl.pallas_call(
        _tc1_body,
        grid=(N // RB,),
        in_specs=[
            pl.BlockSpec((RB, D), lambda i: (i, 0)),
            pl.BlockSpec((D, D), lambda i: (0, 0)),
            pl.BlockSpec((RB, DW), lambda i: (i, 0)),
            pl.BlockSpec((RB, DW), lambda i: (i, 0)),
        ],
        out_specs=[
            pl.BlockSpec((RB, DC), lambda i: (i, 0)),
            pl.BlockSpec((RB, DC), lambda i: (i, 0)),
        ],
        out_shape=[
            jax.ShapeDtypeStruct((N, DC), jnp.float32),
            jax.ShapeDtypeStruct((N, DC), jnp.float32),
        ],
    )(x, W1, dega, degb)


def _tc2(agg, dega, degb, b1, W2):
    return pl.pallas_call(
        _tc2_body,
        grid=(N // RB,),
        in_specs=[
            pl.BlockSpec((1, RB, DC), lambda i: (0, i, 0)),
            pl.BlockSpec((1, RB, DC), lambda i: (1, i, 0)),
            pl.BlockSpec((RB, DW), lambda i: (i, 0)),
            pl.BlockSpec((RB, DW), lambda i: (i, 0)),
            pl.BlockSpec((D,), lambda i: (0,)),
            pl.BlockSpec((D, D), lambda i: (0, 0)),
        ],
        out_specs=[
            pl.BlockSpec((RB, DC), lambda i: (i, 0)),
            pl.BlockSpec((RB, DC), lambda i: (i, 0)),
        ],
        out_shape=[
            jax.ShapeDtypeStruct((N, DC), jnp.float32),
            jax.ShapeDtypeStruct((N, DC), jnp.float32),
        ],
    )(agg, agg, dega, degb, b1, W2)


def _tc3(agg, dega, degb, b2):
    return pl.pallas_call(
        _tc3_body,
        grid=(N // RB,),
        in_specs=[
            pl.BlockSpec((1, RB, DC), lambda i: (0, i, 0)),
            pl.BlockSpec((1, RB, DC), lambda i: (1, i, 0)),
            pl.BlockSpec((RB, DW), lambda i: (i, 0)),
            pl.BlockSpec((RB, DW), lambda i: (i, 0)),
            pl.BlockSpec((D,), lambda i: (0,)),
        ],
        out_specs=pl.BlockSpec((RB, D), lambda i: (i, 0)),
        out_shape=jax.ShapeDtypeStruct((N, D), jnp.float32),
    )(agg, agg, dega, degb, b2)


def kernel(x, edge_index, W1, b1, W2, b2):
    ei = edge_index.astype(jnp.int32)
    dst_d = ei[1].reshape(NW, DNCH, CH)
    src_a = ei[0].reshape(NS, ANCH, CH)
    dst_a = ei[1].reshape(NS, ANCH, CH)

    dego = _deg_kernel(dst_d)
    dega, degb = dego[0], dego[1]

    h1lo, h1hi = _tc1(x, W1, dega, degb)
    a1 = _agg_kernel(h1lo, h1hi, src_a, dst_a)
    h2lo, h2hi = _tc2(a1, dega, degb, b1, W2)
    a2 = _agg_kernel(h2lo, h2hi, src_a, dst_a)
    return _tc3(a2, dega, degb, b2)
